# baseline (speedup 1.0000x reference)
import functools

import jax
import jax.numpy as jnp
from jax.experimental import pallas as pl
from jax.experimental.pallas import tpu as pltpu

_N = 100000
_C = 18
_V = 8
_BL = 32768
_GRID_A = (_N * _C + _BL - 1) // _BL
_BP = 4000
_GRID_B = _N // _BP
_ROWS = 1024
_COLS = 128
_PAD = _ROWS * _COLS
_K = 2048

_STAGE_JS = []
_STAGE_KBS = []
for _k in range(1, 12):
    _j = 1 << (_k - 1)
    while _j >= 1:
        _STAGE_JS.append(_j)
        _STAGE_KBS.append(1 << _k)
        _j //= 2


def _view_kernel(x_ref, y_ref, m8_ref, g8_ref, sums_ref):
    i = pl.program_id(0)
    x = x_ref[...]
    y = y_ref[...]
    m = jnp.max(x, axis=0, keepdims=True)
    g = jnp.max(y, axis=0, keepdims=True)
    m8_ref[0, :, :] = m
    g8_ref[0, :, :] = g
    li = jax.lax.broadcasted_iota(jnp.int32, (1, _BL), 1)
    valid = (li + i * _BL) < (_N * _C)
    pb = jnp.where(jnp.logical_and(m >= 0.0, valid), 1.0, 0.0)
    gv = jnp.where(valid, g, 0.0)
    tp = jnp.sum(pb * gv, axis=(0, 1), keepdims=True)
    p = jnp.sum(pb, axis=(0, 1), keepdims=True)
    gs = jnp.sum(gv, axis=(0, 1), keepdims=True)
    part = jnp.concatenate([tp, p, gs], axis=1)
    sums_ref[...] = jnp.where(i == 0, part, sums_ref[...] + part)


def _point_kernel(mt_ref, gt_ref, ml_ref, lb_ref):
    m = mt_ref[...]
    g = gt_ref[...]
    rowmax = jnp.max(m, axis=1)
    ji = jax.lax.broadcasted_iota(jnp.int32, (_BP, _C), 1)
    cand = jnp.where(m == rowmax[:, None], ji, _C)
    aidx = jnp.min(cand, axis=1)
    lab = jnp.sum(jnp.where(ji == aidx[:, None], g, 0.0), axis=1)
    ml_ref[0, 0, :] = rowmax
    lb_ref[0, 0, :] = lab


def _ce_stage(vd_ref, ld_ref, rows, j, kbit):
    riota = jax.lax.broadcasted_iota(jnp.int32, (rows, _COLS), 0)
    ciota = jax.lax.broadcasted_iota(jnp.int32, (rows, _COLS), 1)
    jr = jnp.right_shift(j, 7)
    low = (jnp.bitwise_and(ciota, jnp.bitwise_and(j, _COLS - 1))
           + jnp.bitwise_and(riota, jr)) == 0
    desc = (jnp.bitwise_and(ciota, jnp.bitwise_and(kbit, _COLS - 1))
            + jnp.bitwise_and(riota, jnp.right_shift(kbit, 7))) == 0

    def lane_stage(_):
        x = vd_ref[pl.ds(0, rows), :]
        xl = ld_ref[pl.ds(0, rows), :]
        up = pltpu.roll(x, _COLS - j, axis=1)
        dn = pltpu.roll(x, j, axis=1)
        lup = pltpu.roll(xl, _COLS - j, axis=1)
        ldn = pltpu.roll(xl, j, axis=1)
        return jnp.where(low, up, dn), jnp.where(low, lup, ldn)

    def row_stage(_):
        up = vd_ref[pl.ds(jr, rows), :]
        dn = vd_ref[pl.ds(rows - jr, rows), :]
        lup = ld_ref[pl.ds(jr, rows), :]
        ldn = ld_ref[pl.ds(rows - jr, rows), :]
        return jnp.where(low, up, dn), jnp.where(low, lup, ldn)

    pv, plab = jax.lax.cond(j < _COLS, lane_stage, row_stage, 0)
    v = vd_ref[pl.ds(0, rows), :]
    l = ld_ref[pl.ds(0, rows), :]
    keep_max = jnp.logical_not(jnp.logical_xor(low, desc))
    take = jnp.logical_or(
        jnp.logical_and(keep_max, pv > v),
        jnp.logical_and(jnp.logical_not(keep_max), pv < v),
    )
    nv = jnp.where(take, pv, v)
    nl = jnp.where(take, plab, l)
    vd_ref[pl.ds(0, rows), :] = nv
    vd_ref[pl.ds(rows, rows), :] = nv
    ld_ref[pl.ds(0, rows), :] = nl
    ld_ref[pl.ds(rows, rows), :] = nl


def _topk_kernel(v_ref, l_ref, sums_ref, js_ref, kb_ref, scal_ref, topv_ref,
                 vd_ref, ld_ref):
    vd_ref[pl.ds(0, _ROWS), :] = v_ref[...]
    vd_ref[pl.ds(_ROWS, _ROWS), :] = v_ref[...]
    ld_ref[pl.ds(0, _ROWS), :] = l_ref[...]
    ld_ref[pl.ds(_ROWS, _ROWS), :] = l_ref[...]

    def p1_body(s, carry):
        _ce_stage(vd_ref, ld_ref, _ROWS, js_ref[s], kb_ref[s])
        return carry

    jax.lax.fori_loop(0, len(_STAGE_JS), p1_body, 0)

    rows = _ROWS
    for rnd in range(6):
        half = rows // 2
        a_v = vd_ref[pl.ds(0, rows), :].reshape(half // 16, 2, 16, _COLS)
        a_l = ld_ref[pl.ds(0, rows), :].reshape(half // 16, 2, 16, _COLS)
        av, bv = a_v[:, 0], a_v[:, 1]
        al, bl = a_l[:, 0], a_l[:, 1]
        take = bv > av
        nv = jnp.where(take, bv, av).reshape(half, _COLS)
        nl = jnp.where(take, bl, al).reshape(half, _COLS)
        rows = half
        vd_ref[pl.ds(0, rows), :] = nv
        vd_ref[pl.ds(rows, rows), :] = nv
        ld_ref[pl.ds(0, rows), :] = nl
        ld_ref[pl.ds(rows, rows), :] = nl
        kbit = (1 << 17) if rnd == 5 else (1 << 11)

        def p2_body(s, carry, rows=rows, kbit=kbit):
            _ce_stage(vd_ref, ld_ref, rows, jnp.left_shift(1, 10 - s), kbit)
            return carry

        jax.lax.fori_loop(0, 11, p2_body, 0)

    vt = vd_ref[pl.ds(0, 16), :]
    lt = ld_ref[pl.ds(0, 16), :]
    topv_ref[...] = jax.nn.sigmoid(vt)
    eq = jnp.logical_not(jnp.logical_xor(vt >= 0.0, lt > 0.5)).astype(jnp.float32)
    tka = jnp.sum(eq, axis=(0, 1), keepdims=True) / float(_K)
    s = sums_ref[...]
    tp = s[:, 0:1]
    p = s[:, 1:2]
    g = s[:, 2:3]
    tot = float(_N * _C)
    acc = (tot - p - g + 2.0 * tp) / tot
    prec = tp / jnp.maximum(p, 1.0)
    rec = tp / jnp.maximum(g, 1.0)
    f1 = 2.0 * tp / jnp.maximum(p + g, 1.0)
    zero = jnp.zeros((1, 3), jnp.float32)
    scal_ref[...] = jnp.concatenate([acc, prec, rec, f1, tka, zero], axis=1)


@jax.jit
def kernel(grasp_labels, grasp_prediction):
    xf = grasp_prediction.reshape(_V, _N * _C)
    yf = grasp_labels.reshape(_V, _N * _C)
    m8, g8, sums = pl.pallas_call(
        _view_kernel,
        grid=(_GRID_A,),
        in_specs=[
            pl.BlockSpec((_V, _BL), lambda i: (0, i)),
            pl.BlockSpec((_V, _BL), lambda i: (0, i)),
        ],
        out_specs=[
            pl.BlockSpec((1, 1, _BL), lambda i: (i, 0, 0)),
            pl.BlockSpec((1, 1, _BL), lambda i: (i, 0, 0)),
            pl.BlockSpec((1, 3), lambda i: (0, 0)),
        ],
        out_shape=[
            jax.ShapeDtypeStruct((_GRID_A, 1, _BL), jnp.float32),
            jax.ShapeDtypeStruct((_GRID_A, 1, _BL), jnp.float32),
            jax.ShapeDtypeStruct((1, 3), jnp.float32),
        ],
    )(xf, yf)
    mt = m8.reshape(_GRID_A * _BL)[: _N * _C].reshape(_N, _C)
    gt = g8.reshape(_GRID_A * _BL)[: _N * _C].reshape(_N, _C)
    ml2, lb2 = pl.pallas_call(
        _point_kernel,
        grid=(_GRID_B,),
        in_specs=[
            pl.BlockSpec((_BP, _C), lambda i: (i, 0)),
            pl.BlockSpec((_BP, _C), lambda i: (i, 0)),
        ],
        out_specs=[
            pl.BlockSpec((1, 1, _BP), lambda i: (i, 0, 0)),
            pl.BlockSpec((1, 1, _BP), lambda i: (i, 0, 0)),
        ],
        out_shape=[
            jax.ShapeDtypeStruct((_GRID_B, 1, _BP), jnp.float32),
            jax.ShapeDtypeStruct((_GRID_B, 1, _BP), jnp.float32),
        ],
    )(mt, gt)
    pad = jnp.full((_PAD - _N,), -jnp.inf, jnp.float32)
    vp = jnp.concatenate([ml2.reshape(_N), pad])
    vp = vp.reshape(_ROWS, _COLS)
    lp = jnp.concatenate([lb2.reshape(_N),
                          jnp.zeros((_PAD - _N,), jnp.float32)])
    lp = lp.reshape(_ROWS, _COLS)
    js = jnp.array(_STAGE_JS, jnp.int32)
    kbs = jnp.array(_STAGE_KBS, jnp.int32)
    scal, topv2 = pl.pallas_call(
        _topk_kernel,
        in_specs=[
            pl.BlockSpec(memory_space=pltpu.VMEM),
            pl.BlockSpec(memory_space=pltpu.VMEM),
            pl.BlockSpec(memory_space=pltpu.VMEM),
            pl.BlockSpec(memory_space=pltpu.SMEM),
            pl.BlockSpec(memory_space=pltpu.SMEM),
        ],
        scratch_shapes=[
            pltpu.VMEM((2 * _ROWS, _COLS), jnp.float32),
            pltpu.VMEM((2 * _ROWS, _COLS), jnp.float32),
        ],
        out_shape=[
            jax.ShapeDtypeStruct((1, 8), jnp.float32),
            jax.ShapeDtypeStruct((16, _COLS), jnp.float32),
        ],
    )(vp, lp, sums, js, kbs)
    topv = topv2.reshape(_K)
    return (scal[0, 0], scal[0, 1], scal[0, 2], scal[0, 3], scal[0, 4], topv)


# speedup vs baseline: 5.6123x; 5.6123x over previous
import functools

import jax
import jax.numpy as jnp
from jax.experimental import pallas as pl
from jax.experimental.pallas import tpu as pltpu

_N = 100000
_C = 18
_V = 8
_BN = 2000
_GRID = _N // _BN
_ROWS = 1024
_COLS = 128
_PAD = _ROWS * _COLS
_K = 2048

_STAGE_JS = []
_STAGE_KBS = []
for _k in range(1, 12):
    _j = 1 << (_k - 1)
    while _j >= 1:
        _STAGE_JS.append(_j)
        _STAGE_KBS.append(1 << _k)
        _j //= 2


def _reduce_kernel(x_ref, y_ref, ml_ref, lb_ref, sums_ref):
    i = pl.program_id(0)
    x = x_ref[...]
    y = y_ref[...]
    m = jnp.max(x, axis=0)
    g = jnp.max(y, axis=0)
    rowmax = jnp.max(m, axis=1)
    ji = jax.lax.broadcasted_iota(jnp.int32, (_BN, _C), 1)
    cand = jnp.where(m == rowmax[:, None], ji, _C)
    aidx = jnp.min(cand, axis=1)
    lab = jnp.sum(jnp.where(ji == aidx[:, None], g, 0.0), axis=1)
    ml_ref[0, 0, :] = rowmax
    lb_ref[0, 0, :] = lab
    pb = (m >= 0.0).astype(jnp.float32)
    tp = jnp.sum(pb * g, axis=(0, 1), keepdims=True)
    p = jnp.sum(pb, axis=(0, 1), keepdims=True)
    gs = jnp.sum(g, axis=(0, 1), keepdims=True)
    part = jnp.concatenate([tp, p, gs], axis=1)
    sums_ref[...] = jnp.where(i == 0, part, sums_ref[...] + part)


def _ce_stage(vd_ref, ld_ref, rows, j, kbit):
    riota = jax.lax.broadcasted_iota(jnp.int32, (rows, _COLS), 0)
    ciota = jax.lax.broadcasted_iota(jnp.int32, (rows, _COLS), 1)
    jr = jnp.right_shift(j, 7)
    low = (jnp.bitwise_and(ciota, jnp.bitwise_and(j, _COLS - 1))
           + jnp.bitwise_and(riota, jr)) == 0
    desc = (jnp.bitwise_and(ciota, jnp.bitwise_and(kbit, _COLS - 1))
            + jnp.bitwise_and(riota, jnp.right_shift(kbit, 7))) == 0

    def lane_stage(_):
        x = vd_ref[pl.ds(0, rows), :]
        xl = ld_ref[pl.ds(0, rows), :]
        up = pltpu.roll(x, _COLS - j, axis=1)
        dn = pltpu.roll(x, j, axis=1)
        lup = pltpu.roll(xl, _COLS - j, axis=1)
        ldn = pltpu.roll(xl, j, axis=1)
        return jnp.where(low, up, dn), jnp.where(low, lup, ldn)

    def row_stage(_):
        up = vd_ref[pl.ds(jr, rows), :]
        dn = vd_ref[pl.ds(rows - jr, rows), :]
        lup = ld_ref[pl.ds(jr, rows), :]
        ldn = ld_ref[pl.ds(rows - jr, rows), :]
        return jnp.where(low, up, dn), jnp.where(low, lup, ldn)

    pv, plab = jax.lax.cond(j < _COLS, lane_stage, row_stage, 0)
    v = vd_ref[pl.ds(0, rows), :]
    l = ld_ref[pl.ds(0, rows), :]
    keep_max = jnp.logical_not(jnp.logical_xor(low, desc))
    take = jnp.logical_or(
        jnp.logical_and(keep_max, pv > v),
        jnp.logical_and(jnp.logical_not(keep_max), pv < v),
    )
    nv = jnp.where(take, pv, v)
    nl = jnp.where(take, plab, l)
    vd_ref[pl.ds(0, rows), :] = nv
    vd_ref[pl.ds(rows, rows), :] = nv
    ld_ref[pl.ds(0, rows), :] = nl
    ld_ref[pl.ds(rows, rows), :] = nl


def _topk_kernel(v_ref, l_ref, sums_ref, js_ref, kb_ref, scal_ref, topv_ref,
                 vd_ref, ld_ref):
    vd_ref[pl.ds(0, _ROWS), :] = v_ref[...]
    vd_ref[pl.ds(_ROWS, _ROWS), :] = v_ref[...]
    ld_ref[pl.ds(0, _ROWS), :] = l_ref[...]
    ld_ref[pl.ds(_ROWS, _ROWS), :] = l_ref[...]

    def p1_body(s, carry):
        _ce_stage(vd_ref, ld_ref, _ROWS, js_ref[s], kb_ref[s])
        return carry

    jax.lax.fori_loop(0, len(_STAGE_JS), p1_body, 0)

    rows = _ROWS
    for rnd in range(6):
        half = rows // 2
        a_v = vd_ref[pl.ds(0, rows), :].reshape(half // 16, 2, 16, _COLS)
        a_l = ld_ref[pl.ds(0, rows), :].reshape(half // 16, 2, 16, _COLS)
        av, bv = a_v[:, 0], a_v[:, 1]
        al, bl = a_l[:, 0], a_l[:, 1]
        take = bv > av
        nv = jnp.where(take, bv, av).reshape(half, _COLS)
        nl = jnp.where(take, bl, al).reshape(half, _COLS)
        rows = half
        vd_ref[pl.ds(0, rows), :] = nv
        vd_ref[pl.ds(rows, rows), :] = nv
        ld_ref[pl.ds(0, rows), :] = nl
        ld_ref[pl.ds(rows, rows), :] = nl
        kbit = (1 << 17) if rnd == 5 else (1 << 11)

        def p2_body(s, carry, rows=rows, kbit=kbit):
            _ce_stage(vd_ref, ld_ref, rows, jnp.left_shift(1, 10 - s), kbit)
            return carry

        jax.lax.fori_loop(0, 11, p2_body, 0)

    vt = vd_ref[pl.ds(0, 16), :]
    lt = ld_ref[pl.ds(0, 16), :]
    topv_ref[...] = jax.nn.sigmoid(vt)
    eq = jnp.logical_not(jnp.logical_xor(vt >= 0.0, lt > 0.5)).astype(jnp.float32)
    tka = jnp.sum(eq, axis=(0, 1), keepdims=True) / float(_K)
    s = sums_ref[...]
    tp = s[:, 0:1]
    p = s[:, 1:2]
    g = s[:, 2:3]
    tot = float(_N * _C)
    acc = (tot - p - g + 2.0 * tp) / tot
    prec = tp / jnp.maximum(p, 1.0)
    rec = tp / jnp.maximum(g, 1.0)
    f1 = 2.0 * tp / jnp.maximum(p + g, 1.0)
    zero = jnp.zeros((1, 3), jnp.float32)
    scal_ref[...] = jnp.concatenate([acc, prec, rec, f1, tka, zero], axis=1)


@jax.jit
def kernel(grasp_labels, grasp_prediction):
    x = grasp_prediction
    y = grasp_labels
    ml, lb, sums = pl.pallas_call(
        _reduce_kernel,
        grid=(_GRID,),
        in_specs=[
            pl.BlockSpec((_V, _BN, _C), lambda i: (0, i, 0)),
            pl.BlockSpec((_V, _BN, _C), lambda i: (0, i, 0)),
        ],
        out_specs=[
            pl.BlockSpec((1, 1, _BN), lambda i: (i, 0, 0)),
            pl.BlockSpec((1, 1, _BN), lambda i: (i, 0, 0)),
            pl.BlockSpec((1, 3), lambda i: (0, 0)),
        ],
        out_shape=[
            jax.ShapeDtypeStruct((_GRID, 1, _BN), jnp.float32),
            jax.ShapeDtypeStruct((_GRID, 1, _BN), jnp.float32),
            jax.ShapeDtypeStruct((1, 3), jnp.float32),
        ],
    )(x, y)
    pad = jnp.full((_PAD - _N,), -jnp.inf, jnp.float32)
    vp = jnp.concatenate([ml.reshape(_N), pad]).reshape(_ROWS, _COLS)
    lp = jnp.concatenate([lb.reshape(_N), jnp.zeros((_PAD - _N,), jnp.float32)])
    lp = lp.reshape(_ROWS, _COLS)
    js = jnp.array(_STAGE_JS, jnp.int32)
    kbs = jnp.array(_STAGE_KBS, jnp.int32)
    scal, topv2 = pl.pallas_call(
        _topk_kernel,
        in_specs=[
            pl.BlockSpec(memory_space=pltpu.VMEM),
            pl.BlockSpec(memory_space=pltpu.VMEM),
            pl.BlockSpec(memory_space=pltpu.VMEM),
            pl.BlockSpec(memory_space=pltpu.SMEM),
            pl.BlockSpec(memory_space=pltpu.SMEM),
        ],
        scratch_shapes=[
            pltpu.VMEM((2 * _ROWS, _COLS), jnp.float32),
            pltpu.VMEM((2 * _ROWS, _COLS), jnp.float32),
        ],
        out_shape=[
            jax.ShapeDtypeStruct((1, 8), jnp.float32),
            jax.ShapeDtypeStruct((16, _COLS), jnp.float32),
        ],
    )(vp, lp, sums, js, kbs)
    topv = topv2.reshape(_K)
    return (scal[0, 0], scal[0, 1], scal[0, 2], scal[0, 3], scal[0, 4], topv)
